# baseline (speedup 1.0000x reference)
import functools

import jax
import jax.numpy as jnp
from jax import lax
from jax.experimental import pallas as pl
from jax.experimental.pallas import tpu as pltpu
from jax.experimental.pallas import tpu_sc as plsc

N = 10000
D = 128
LANES = 16
NC = 2
NS = 16
NW = NC * NS

K1 = 80
N1 = 125
N2 = 63
NSINK = 8
ZR = 16


def _sc_aggregate(x, pk1, ea4, pk2):
    mesh = plsc.VectorSubcoreMesh(core_axis_name="c", subcore_axis_name="s")

    @functools.partial(
        pl.kernel,
        out_type=(
            jax.ShapeDtypeStruct((NC, N, D), jnp.float32),
            jax.ShapeDtypeStruct((NC, N, D), jnp.float32),
        ),
        mesh=mesh,
        scratch_types=[
            pltpu.VMEM_SHARED((N + NSINK, D), jnp.float32),
            pltpu.VMEM((ZR, D), jnp.float32),
            pltpu.VMEM((N1, K1), jnp.int32),
            pltpu.VMEM((K1,), jnp.int32),
            pltpu.VMEM((K1,), jnp.int32),
            pltpu.VMEM((K1,), jnp.int32),
            pltpu.VMEM((K1,), jnp.int32),
            pltpu.VMEM((K1, D), jnp.float32),
            pltpu.VMEM((K1, D), jnp.float32),
            pltpu.VMEM((K1, D), jnp.float32),
            pltpu.SemaphoreType.DMA,
            pltpu.SemaphoreType.DMA,
            pltpu.SemaphoreType.DMA,
            pltpu.SemaphoreType.DMA,
            pltpu.SemaphoreType.DMA,
            pltpu.SemaphoreType.DMA,
        ],
    )
    def sc_kernel(x_hbm, pk1_hbm, ea_hbm, pk2_hbm,
                  aggr_out, uag_out, acc, zbuf, pks,
                  sidx0, sidx1, didx0, didx1, xg0, xg1, eab,
                  psem, gsem0, gsem1, esem, ssem0, ssem1):
        c = lax.axis_index("c")
        s = lax.axis_index("s")
        wid = c * NS + s
        row0 = s * 640
        nz = jnp.where(s < NS - 1, 640 // ZR, 400 // ZR)

        cp_pk = pltpu.async_copy(pk1_hbm.at[wid], pks, psem)

        zero = jnp.zeros((LANES,), jnp.float32)

        @pl.loop(0, ZR)
        def _(i):
            for j in range(D // LANES):
                zbuf[i, pl.ds(j * LANES, LANES)] = zero

        def zero_acc():
            @pl.loop(0, nz)
            def _(k):
                pltpu.sync_copy(zbuf, acc.at[pl.ds(row0 + k * ZR, ZR)])

        def copy_out(out_hbm):
            @pl.when(s < NS - 1)
            def _():
                pltpu.sync_copy(acc.at[pl.ds(row0, 640)],
                                out_hbm.at[c, pl.ds(row0, 640)])

            @pl.when(s == NS - 1)
            def _():
                pltpu.sync_copy(acc.at[pl.ds(row0, 400)],
                                out_hbm.at[c, pl.ds(row0, 400)])

        def unpack(g, sb, db):
            for j in range(K1 // LANES):
                sl = pl.ds(j * LANES, LANES)
                v = pks[g, sl]
                sb[sl] = v & 0xFFFF
                db[sl] = v >> 16

        zero_acc()
        cp_pk.wait()
        plsc.subcore_barrier()

        unpack(0, sidx0, didx0)
        pltpu.async_copy(x_hbm.at[sidx0], xg0, gsem0)
        pltpu.async_copy(ea_hbm.at[wid, 0], eab, esem)

        def step1(g, n_ch, xgb, sb, db, gsem, ssem,
                  oxg, osb, odb, ogsem, ossem, with_ea):
            @pl.when(g > 0)
            def _():
                pltpu.make_async_copy(oxg, acc.at[odb], ossem).wait()

            @pl.when(g + 1 < n_ch)
            def _():
                unpack(g + 1, osb, odb)
                pltpu.async_copy(x_hbm.at[osb], oxg, ogsem)

            pltpu.make_async_copy(x_hbm.at[sb], xgb, gsem).wait()
            if with_ea:
                pltpu.make_async_copy(ea_hbm.at[wid, g], eab, esem).wait()

                @plsc.parallel_loop(0, K1, unroll=2)
                def _(i):
                    for j in range(D // LANES):
                        sl = pl.ds(j * LANES, LANES)
                        xgb[i, sl] = jnp.maximum(xgb[i, sl] + eab[i, sl], 0.0)

                @pl.when(g + 1 < n_ch)
                def _():
                    pltpu.async_copy(ea_hbm.at[wid, g + 1], eab, esem)

            pltpu.async_copy(xgb, acc.at[db], ssem, add=True)

        def run_phase(n_ch, with_ea):
            @pl.loop(0, n_ch // 2)
            def _(t):
                g = 2 * t
                step1(g, n_ch, xg0, sidx0, didx0, gsem0, ssem0,
                      xg1, sidx1, didx1, gsem1, ssem1, with_ea)
                step1(g + 1, n_ch, xg1, sidx1, didx1, gsem1, ssem1,
                      xg0, sidx0, didx0, gsem0, ssem0, with_ea)

            step1(n_ch - 1, n_ch, xg0, sidx0, didx0, gsem0, ssem0,
                  xg1, sidx1, didx1, gsem1, ssem1, with_ea)
            pltpu.make_async_copy(xg0, acc.at[didx0], ssem0).wait()

        run_phase(N1, True)
        ucp = pltpu.async_copy(pk2_hbm.at[wid], pks.at[pl.ds(0, N2)], psem)
        plsc.subcore_barrier()
        copy_out(aggr_out)
        zero_acc()
        ucp.wait()
        plsc.subcore_barrier()

        unpack(0, sidx0, didx0)
        pltpu.async_copy(x_hbm.at[sidx0], xg0, gsem0)
        run_phase(N2, False)
        plsc.subcore_barrier()
        copy_out(uag_out)

    return sc_kernel(x, pk1, ea4, pk2)


def _dense_body(sv, x_ref, ap, up, w1t, b1, g1, bt1, w2t, b2, wrelt, brel,
                wroott, gout, btout, o_ref):
    one_eps = sv[0]
    a1 = sv[1]
    a2 = sv[2]
    x = x_ref[...]
    aggr = ap[0] + ap[1]
    h = one_eps * x + aggr
    h = jnp.dot(h, w1t[...], preferred_element_type=jnp.float32) + b1[...]
    mean = jnp.mean(h, axis=0, keepdims=True)
    var = jnp.mean((h - mean) ** 2, axis=0, keepdims=True)
    h = (h - mean) * lax.rsqrt(var + 1e-5) * g1[...] + bt1[...]
    h = jnp.maximum(h, 0.0)
    hd = jnp.dot(h, w2t[...], preferred_element_type=jnp.float32) + b2[...]
    uag = up[0] + up[1]
    hu = (jnp.dot(uag, wrelt[...], preferred_element_type=jnp.float32)
          + brel[...]
          + jnp.dot(x, wroott[...], preferred_element_type=jnp.float32))
    out = x + a1 * hd + a2 * hu
    mean2 = jnp.mean(out, axis=0, keepdims=True)
    var2 = jnp.mean((out - mean2) ** 2, axis=0, keepdims=True)
    out = (out - mean2) * lax.rsqrt(var2 + 1e-5) * gout[...] + btout[...]
    o_ref[...] = jnp.maximum(out, 0.0)


def kernel(x, edge_index, edge_attr_emb, v_idx, eps, W1, b1, g1, bt1, W2, b2,
           Wrel, brel, Wroot, g_out, bt_out, alpha1, alpha2):
    pk1 = (edge_index[0] | (edge_index[1] << 16)).reshape(NW, N1, K1)
    ea4 = edge_attr_emb.reshape(NW, N1, K1, D)
    upad = N2 * K1 - v_idx.shape[1] // NW
    pk2 = jnp.concatenate(
        [(v_idx[0] | (v_idx[1] << 16)).reshape(NW, -1),
         jnp.full((NW, upad), N << 16, jnp.int32)], axis=1).reshape(
             NW, N2, K1)

    ap, up = _sc_aggregate(x, pk1, ea4, pk2)

    sv = jnp.stack([1.0 + eps, alpha1, alpha2])
    row = lambda v: v.reshape(1, D)
    grid_spec = pl.GridSpec(
        in_specs=[pl.BlockSpec(memory_space=pltpu.SMEM)]
        + [pl.BlockSpec(memory_space=pltpu.VMEM)] * 14,
        out_specs=pl.BlockSpec(memory_space=pltpu.VMEM),
    )
    out = pl.pallas_call(
        _dense_body,
        grid_spec=grid_spec,
        out_shape=jax.ShapeDtypeStruct((x.shape[0], D), jnp.float32),
    )(sv, x, ap, up, W1.T, row(b1), row(g1), row(bt1), W2.T, row(b2),
      Wrel.T, row(brel), Wroot.T, row(g_out), row(bt_out))
    return out


# speedup vs baseline: 1.1885x; 1.1885x over previous
import functools

import jax
import jax.numpy as jnp
from jax import lax
from jax.experimental import pallas as pl
from jax.experimental.pallas import tpu as pltpu
from jax.experimental.pallas import tpu_sc as plsc

N = 10000
D = 128
LANES = 16
NC = 2
NS = 16
NW = NC * NS

K1 = 80
N1 = 125
N2 = 63
NSINK = 8
SK = 40
Q1 = 250
Q2 = 126


def _sc_aggregate(x, pk1, ea4, pk2, zrows):
    mesh = plsc.VectorSubcoreMesh(core_axis_name="c", subcore_axis_name="s")

    @functools.partial(
        pl.kernel,
        out_type=(
            jax.ShapeDtypeStruct((NC, N, D), jnp.float32),
            jax.ShapeDtypeStruct((NC, N, D), jnp.float32),
        ),
        mesh=mesh,
        scratch_types=[
            pltpu.VMEM_SHARED((N + NSINK, D), jnp.float32),
            pltpu.VMEM((N1, K1), jnp.int32),
            pltpu.VMEM((6, SK), jnp.int32),
            pltpu.VMEM((6, SK), jnp.int32),
            pltpu.VMEM((2 * SK, D), jnp.float32),
            pltpu.VMEM((2 * SK, D), jnp.float32),
            pltpu.VMEM((2 * SK, D), jnp.float32),
            [pltpu.SemaphoreType.DMA] * 6,
            [pltpu.SemaphoreType.DMA] * 6,
            [pltpu.SemaphoreType.DMA] * 2,
            pltpu.SemaphoreType.DMA,
        ],
    )
    def sc_kernel(x_hbm, pk1_hbm, ea_hbm, pk2_hbm, z_hbm,
                  aggr_out, uag_out, acc, pks, sidx, didx,
                  xga, xgb, eab, gsems, ssems, esems, psem):
        c = lax.axis_index("c")
        s = lax.axis_index("s")
        wid = c * NS + s
        row0 = s * 640

        B = [xga.at[pl.ds(0, SK)], xga.at[pl.ds(SK, SK)],
             xgb.at[pl.ds(0, SK)], xgb.at[pl.ds(SK, SK)],
             eab.at[pl.ds(0, SK)], eab.at[pl.ds(SK, SK)]]
        E = [eab.at[pl.ds(0, SK)], eab.at[pl.ds(SK, SK)]]
        SI = [sidx.at[0], sidx.at[1], sidx.at[2],
              sidx.at[3], sidx.at[4], sidx.at[5]]
        DI = [didx.at[0], didx.at[1], didx.at[2],
              didx.at[3], didx.at[4], didx.at[5]]
        GS = list(gsems)
        SS = list(ssems)
        ES = list(esems)

        cp_pk = pltpu.async_copy(pk1_hbm.at[wid], pks, psem)

        def zero_acc():
            @pl.when(s < NS - 1)
            def _():
                pltpu.sync_copy(z_hbm, acc.at[pl.ds(row0, 640)])

            @pl.when(s == NS - 1)
            def _():
                pltpu.sync_copy(z_hbm.at[pl.ds(0, 400)],
                                acc.at[pl.ds(row0, 400)])

        def copy_out(out_hbm):
            @pl.when(s < NS - 1)
            def _():
                pltpu.sync_copy(acc.at[pl.ds(row0, 640)],
                                out_hbm.at[c, pl.ds(row0, 640)])

            @pl.when(s == NS - 1)
            def _():
                pltpu.sync_copy(acc.at[pl.ds(row0, 400)],
                                out_hbm.at[c, pl.ds(row0, 400)])

        def unpack(q, half, slot):
            row = q // 2
            for off in (0, LANES, SK - LANES):
                v = pks[row, pl.ds(half * SK + off, LANES)]
                SI[slot][pl.ds(off, LANES)] = v & 0xFFFF
                DI[slot][pl.ds(off, LANES)] = v >> 16

        def issue_gather(q, half, slot):
            unpack(q, half, slot)
            pltpu.async_copy(x_hbm.at[SI[slot]], B[slot], GS[slot])

        def wait_gather(slot):
            pltpu.make_async_copy(x_hbm.at[SI[slot]], B[slot],
                                  GS[slot]).wait()

        def issue_scatter(slot):
            pltpu.async_copy(B[slot], acc.at[DI[slot]], SS[slot], add=True)

        def wait_scatter(slot):
            pltpu.make_async_copy(B[slot], acc.at[DI[slot]],
                                  SS[slot]).wait()

        def issue_ea(q, par):
            pltpu.async_copy(ea_hbm.at[wid, q // 2,
                                       pl.ds((q % 2) * SK, SK)],
                             E[par], ES[par])

        def wait_ea(par):
            pltpu.make_async_copy(ea_hbm.at[wid, 0, pl.ds(0, SK)],
                                  E[par], ES[par]).wait()

        zero_acc()
        cp_pk.wait()
        plsc.subcore_barrier()

        for q in range(3):
            issue_gather(q, q % 2, q % 4)
        issue_ea(0, 0)
        issue_ea(1, 1)

        def sub1(q, k):
            @pl.when(q > 0)
            def _():
                wait_scatter((k - 1) % 4)

            @pl.when(q + 3 < Q1)
            def _():
                issue_gather(q + 3, (k + 3) % 2, (k + 3) % 4)

            wait_gather(k)
            wait_ea(k % 2)

            @plsc.parallel_loop(0, SK, unroll=2)
            def _(i):
                for j in range(D // LANES):
                    sl = pl.ds(j * LANES, LANES)
                    B[k][i, sl] = jnp.maximum(B[k][i, sl] + E[k % 2][i, sl],
                                              0.0)

            @pl.when(q + 2 < Q1)
            def _():
                issue_ea(q + 2, k % 2)

            issue_scatter(k)

        @pl.loop(0, (Q1 - 2) // 4)
        def _(t):
            q = 4 * t
            for k in range(4):
                sub1(q + k, k)

        sub1(Q1 - 2, 0)
        sub1(Q1 - 1, 1)
        wait_scatter(1)
        ucp = pltpu.async_copy(pk2_hbm.at[wid], pks.at[pl.ds(0, N2)], psem)
        plsc.subcore_barrier()
        copy_out(aggr_out)
        zero_acc()
        ucp.wait()
        plsc.subcore_barrier()

        for q in range(5):
            issue_gather(q, q % 2, q)

        def sub2(q, k):
            @pl.when(q > 0)
            def _():
                wait_scatter((k - 1) % 6)

            @pl.when(q + 5 < Q2)
            def _():
                issue_gather(q + 5, (k + 5) % 2, (k + 5) % 6)

            wait_gather(k)
            issue_scatter(k)

        @pl.loop(0, Q2 // 6)
        def _(t):
            q = 6 * t
            for k in range(6):
                sub2(q + k, k)

        wait_scatter((Q2 - 1) % 6)
        plsc.subcore_barrier()
        copy_out(uag_out)

    return sc_kernel(x, pk1, ea4, pk2, zrows)


def _dense_body(sv, x_ref, ap, up, w1t, b1, g1, bt1, w2t, b2, wrelt, brel,
                wroott, gout, btout, o_ref):
    one_eps = sv[0]
    a1 = sv[1]
    a2 = sv[2]
    x = x_ref[...]
    aggr = ap[0] + ap[1]
    h = one_eps * x + aggr
    h = jnp.dot(h, w1t[...], preferred_element_type=jnp.float32) + b1[...]
    mean = jnp.mean(h, axis=0, keepdims=True)
    var = jnp.mean((h - mean) ** 2, axis=0, keepdims=True)
    h = (h - mean) * lax.rsqrt(var + 1e-5) * g1[...] + bt1[...]
    h = jnp.maximum(h, 0.0)
    hd = jnp.dot(h, w2t[...], preferred_element_type=jnp.float32) + b2[...]
    uag = up[0] + up[1]
    hu = (jnp.dot(uag, wrelt[...], preferred_element_type=jnp.float32)
          + brel[...]
          + jnp.dot(x, wroott[...], preferred_element_type=jnp.float32))
    out = x + a1 * hd + a2 * hu
    mean2 = jnp.mean(out, axis=0, keepdims=True)
    var2 = jnp.mean((out - mean2) ** 2, axis=0, keepdims=True)
    out = (out - mean2) * lax.rsqrt(var2 + 1e-5) * gout[...] + btout[...]
    o_ref[...] = jnp.maximum(out, 0.0)


def kernel(x, edge_index, edge_attr_emb, v_idx, eps, W1, b1, g1, bt1, W2, b2,
           Wrel, brel, Wroot, g_out, bt_out, alpha1, alpha2):
    pk1 = (edge_index[0] | (edge_index[1] << 16)).reshape(NW, N1, K1)
    ea4 = edge_attr_emb.reshape(NW, N1, K1, D)
    upad = N2 * K1 - v_idx.shape[1] // NW
    pk2 = jnp.concatenate(
        [(v_idx[0] | (v_idx[1] << 16)).reshape(NW, -1),
         jnp.full((NW, upad), N << 16, jnp.int32)], axis=1).reshape(
             NW, N2, K1)

    ap, up = _sc_aggregate(x, pk1, ea4, pk2, jnp.zeros((640, D), jnp.float32))

    sv = jnp.stack([1.0 + eps, alpha1, alpha2])
    row = lambda v: v.reshape(1, D)
    grid_spec = pl.GridSpec(
        in_specs=[pl.BlockSpec(memory_space=pltpu.SMEM)]
        + [pl.BlockSpec(memory_space=pltpu.VMEM)] * 14,
        out_specs=pl.BlockSpec(memory_space=pltpu.VMEM),
    )
    out = pl.pallas_call(
        _dense_body,
        grid_spec=grid_spec,
        out_shape=jax.ShapeDtypeStruct((x.shape[0], D), jnp.float32),
    )(sv, x, ap, up, W1.T, row(b1), row(g1), row(bt1), W2.T, row(b2),
      Wrel.T, row(brel), Wroot.T, row(g_out), row(bt_out))
    return out
